# baseline (speedup 1.0000x reference)
import functools

import jax
import jax.numpy as jnp
from jax.experimental import pallas as pl
from jax.experimental.pallas import tpu as pltpu

_K_BLK = 3072
_NS = 2


def _mm_body(*refs, k_total, nk):
    xt_refs = refs[:_NS]
    wt_ref = refs[_NS]
    o_ref = refs[_NS + 1]
    k = pl.program_id(0)

    @pl.when(k == 0)
    def _init():
        o_ref[...] = jnp.zeros_like(o_ref)

    def contract(xb, wb):
        return jax.lax.dot_general(
            xb, wb, (((0,), (1,)), ((), ())),
            preferred_element_type=jnp.float32,
        )

    @pl.when(k < nk - 1)
    def _full():
        acc = contract(xt_refs[0][...], wt_ref[:, 0:_K_BLK])
        for i in range(1, _NS):
            acc += contract(
                xt_refs[i][...], wt_ref[:, i * _K_BLK:(i + 1) * _K_BLK]
            )
        o_ref[...] += acc

    @pl.when(k == nk - 1)
    def _tail():
        base = (nk - 1) * _NS * _K_BLK
        acc = None
        for i in range(_NS):
            rem = k_total - (base + i * _K_BLK)
            if rem <= 0:
                continue
            xb = xt_refs[i][...]
            wb = wt_ref[:, i * _K_BLK:(i + 1) * _K_BLK]
            if rem < _K_BLK:
                row = jax.lax.broadcasted_iota(jnp.int32, xb.shape, 0)
                xb = jnp.where(row < rem, xb, 0.0)
                col = jax.lax.broadcasted_iota(jnp.int32, wb.shape, 1)
                wb = jnp.where(col < rem, wb, 0.0)
            part = contract(xb, wb)
            acc = part if acc is None else acc + part
        o_ref[...] += acc


def kernel(x, W):
    m, k_total = x.shape
    _, n = W.shape
    span = _NS * _K_BLK
    nk = pl.cdiv(k_total, span)
    nblk = pl.cdiv(k_total, _K_BLK)
    xt = jnp.transpose(x)
    wt = jnp.transpose(W)

    def x_spec(i):
        return pl.BlockSpec(
            (_K_BLK, m),
            lambda k, i=i: (jnp.minimum(k * _NS + i, nblk - 1), 0),
        )

    return pl.pallas_call(
        functools.partial(_mm_body, k_total=k_total, nk=nk),
        grid=(nk,),
        in_specs=[x_spec(i) for i in range(_NS)]
        + [pl.BlockSpec((n, span), lambda k: (0, k))],
        out_specs=pl.BlockSpec((m, n), lambda k: (0, 0)),
        out_shape=jax.ShapeDtypeStruct((m, n), jnp.float32),
        compiler_params=pltpu.CompilerParams(
            dimension_semantics=("arbitrary",),
        ),
    )(*([xt] * _NS), wt)


# speedup vs baseline: 1.0283x; 1.0283x over previous
import functools

import jax
import jax.numpy as jnp
from jax.experimental import pallas as pl
from jax.experimental.pallas import tpu as pltpu

_K_BLK = 3072


def _mm_body(xt_ref, wt_ref, o_ref, *, k_total, nk):
    k = pl.program_id(0)

    def contract(xb, wb):
        return jax.lax.dot_general(
            xb, wb, (((0,), (1,)), ((), ())),
            preferred_element_type=jnp.float32,
        )

    @pl.when(k == 0)
    def _first():
        o_ref[...] = contract(xt_ref[...], wt_ref[...])

    @pl.when(jnp.logical_and(k > 0, k < nk - 1))
    def _full():
        o_ref[...] += contract(xt_ref[...], wt_ref[...])

    @pl.when(k == nk - 1)
    def _tail():
        rem = k_total - (nk - 1) * _K_BLK
        xb = xt_ref[...]
        row = jax.lax.broadcasted_iota(jnp.int32, xb.shape, 0)
        xb = jnp.where(row < rem, xb, 0.0)
        wb = wt_ref[...]
        col = jax.lax.broadcasted_iota(jnp.int32, wb.shape, 1)
        wb = jnp.where(col < rem, wb, 0.0)
        o_ref[...] += contract(xb, wb)


def kernel(x, W):
    m, k_total = x.shape
    _, n = W.shape
    nk = pl.cdiv(k_total, _K_BLK)
    xt = jnp.transpose(x)
    wt = jnp.transpose(W)
    return pl.pallas_call(
        functools.partial(_mm_body, k_total=k_total, nk=nk),
        grid=(nk,),
        in_specs=[
            pl.BlockSpec((_K_BLK, m), lambda k: (k, 0)),
            pl.BlockSpec((n, _K_BLK), lambda k: (0, k)),
        ],
        out_specs=pl.BlockSpec((m, n), lambda k: (0, 0)),
        out_shape=jax.ShapeDtypeStruct((m, n), jnp.float32),
        compiler_params=pltpu.CompilerParams(
            dimension_semantics=("arbitrary",),
        ),
    )(xt, wt)


# speedup vs baseline: 1.0333x; 1.0049x over previous
import functools

import jax
import jax.numpy as jnp
from jax.experimental import pallas as pl
from jax.experimental.pallas import tpu as pltpu

_K_BLK = 3072


def _mm_body(xt_ref, wt_ref, o_ref, *, k_total, nk):
    k = pl.program_id(0)

    def contract(xb, wb):
        return jax.lax.dot_general(
            xb, wb, (((0,), (1,)), ((), ())),
            preferred_element_type=jnp.float32,
        )

    @pl.when(k == 0)
    def _first():
        o_ref[...] = contract(xt_ref[...], wt_ref[...])

    @pl.when(jnp.logical_and(k > 0, k < nk - 1))
    def _full():
        o_ref[...] += contract(xt_ref[...], wt_ref[...])

    @pl.when(k == nk - 1)
    def _tail():
        rem = k_total - (nk - 1) * _K_BLK
        o_ref[...] += contract(xt_ref[0:rem, :], wt_ref[:, 0:rem])


def kernel(x, W):
    m, k_total = x.shape
    _, n = W.shape
    nk = pl.cdiv(k_total, _K_BLK)
    xt = jnp.transpose(x)
    wt = jnp.transpose(W)
    return pl.pallas_call(
        functools.partial(_mm_body, k_total=k_total, nk=nk),
        grid=(nk,),
        in_specs=[
            pl.BlockSpec((_K_BLK, m), lambda k: (k, 0)),
            pl.BlockSpec((n, _K_BLK), lambda k: (0, k)),
        ],
        out_specs=pl.BlockSpec((m, n), lambda k: (0, 0)),
        out_shape=jax.ShapeDtypeStruct((m, n), jnp.float32),
        compiler_params=pltpu.CompilerParams(
            dimension_semantics=("arbitrary",),
        ),
    )(xt, wt)
